# baseline (speedup 1.0000x reference)
import functools

import jax
import jax.numpy as jnp
from jax import lax
from jax.experimental import pallas as pl
from jax.experimental.pallas import tpu as pltpu
from jax.experimental.pallas import tpu_sc as plsc

_LANES = 16
_GRP = 8


def _make_lookup(V, D, B):
    info = plsc.get_sparse_core_info()
    NC, NS = info.num_cores, info.num_subcores
    NW = NC * NS
    assert B % (128 * NW) == 0
    b_per_w = B // NW
    n_chunks = b_per_w // 128
    n_grp = 128 // _GRP

    mesh = plsc.VectorSubcoreMesh(core_axis_name="c", subcore_axis_name="s")

    @functools.partial(
        pl.kernel,
        mesh=mesh,
        out_type=jax.ShapeDtypeStruct((D, B), jnp.float32),
        scratch_types=[
            pltpu.VMEM((b_per_w,), jnp.int32),
            pltpu.VMEM((2, _GRP, D, 128), jnp.float32),
            pltpu.VMEM((D, 128), jnp.float32),
            pltpu.SemaphoreType.DMA,
            pltpu.SemaphoreType.DMA,
            pltpu.SemaphoreType.DMA,
        ],
        compiler_params=pltpu.CompilerParams(
            use_tc_tiling_on_sc=True, needs_layout_passes=False
        ),
    )
    def k(tab_hbm, idx_hbm, out_hbm, idx_v, blk_v, col_v, sem_i, s0, s1):
        wid = lax.axis_index("s") * NC + lax.axis_index("c")
        base = wid * b_per_w
        pltpu.async_copy(idx_hbm.at[pl.ds(base, b_per_w)], idx_v, sem_i).wait()

        rows0 = lax.iota(jnp.int32, _LANES)
        rows1 = rows0 + _LANES
        sems = (s0, s1)

        def fire(c, g):
            vec = idx_v[pl.ds(c * 128 + (g // 2) * _LANES, _LANES)]
            half = (g % 2) * _GRP
            for j in range(_GRP):
                boff = pl.multiple_of((vec[half + j] >> 7) * 128, 128)
                for a in range(D // 8):
                    pltpu.async_copy(
                        tab_hbm.at[pl.ds(8 * a, 8), pl.ds(boff, 128)],
                        blk_v.at[g % 2, j, pl.ds(8 * a, 8)],
                        sems[g % 2],
                    )

        def drain_extract(c, g):
            for _ in range(_GRP):
                pltpu.make_async_copy(
                    tab_hbm.at[:, pl.ds(0, 128)],
                    blk_v.at[0, 0],
                    sems[g % 2],
                ).wait()
            vec = idx_v[pl.ds(c * 128 + (g // 2) * _LANES, _LANES)]
            half = (g % 2) * _GRP
            lanes = jnp.bitwise_and(vec, 127)
            for j in range(_GRP):
                lane = jnp.broadcast_to(lanes[half + j], (_LANES,))
                pos = jnp.broadcast_to(g * _GRP + j, (_LANES,))
                c0 = plsc.load_gather(blk_v.at[g % 2, j], [rows0, lane])
                c1 = plsc.load_gather(blk_v.at[g % 2, j], [rows1, lane])
                plsc.store_scatter(col_v, [rows0, pos], c0)
                plsc.store_scatter(col_v, [rows1, pos], c1)

        def chunk_body(c, carry):
            fire(c, 0)
            for g in range(n_grp):
                if g + 1 < n_grp:
                    fire(c, g + 1)
                drain_extract(c, g)
            ob = pl.multiple_of(base + c * 128, 128)
            pltpu.sync_copy(col_v, out_hbm.at[:, pl.ds(ob, 128)])
            return carry

        lax.fori_loop(0, n_chunks, chunk_body, 0)

    return k


def kernel(indices, table):
    B, = indices.shape
    V, D = table.shape
    lookup = _make_lookup(V, D, B)
    out_t = lookup(table.T, indices.astype(jnp.int32))
    return out_t.T


# speedup vs baseline: 1.0029x; 1.0029x over previous
import functools

import jax
import jax.numpy as jnp
from jax import lax
from jax.experimental import pallas as pl
from jax.experimental.pallas import tpu as pltpu
from jax.experimental.pallas import tpu_sc as plsc

_LANES = 16
_GRP = 8


def _make_lookup(V, D, B):
    info = plsc.get_sparse_core_info()
    NC, NS = info.num_cores, info.num_subcores
    NW = NC * NS
    assert B % (128 * NW) == 0
    b_per_w = B // NW
    n_chunks = b_per_w // 128
    n_grp = 128 // _GRP

    mesh = plsc.VectorSubcoreMesh(core_axis_name="c", subcore_axis_name="s")

    @functools.partial(
        pl.kernel,
        mesh=mesh,
        out_type=jax.ShapeDtypeStruct((D, B), jnp.float32),
        scratch_types=[
            pltpu.VMEM((b_per_w,), jnp.int32),
            pltpu.VMEM((2, _GRP, D, 128), jnp.float32),
            pltpu.VMEM((D, 128), jnp.float32),
            pltpu.SemaphoreType.DMA,
            pltpu.SemaphoreType.DMA,
            pltpu.SemaphoreType.DMA,
        ],
        compiler_params=pltpu.CompilerParams(
            use_tc_tiling_on_sc=True, needs_layout_passes=False
        ),
    )
    def k(tab_hbm, idx_hbm, out_hbm, idx_v, blk_v, col_v, sem_i, s0, s1):
        wid = lax.axis_index("s") * NC + lax.axis_index("c")
        base = wid * b_per_w
        pltpu.async_copy(idx_hbm.at[pl.ds(base, b_per_w)], idx_v, sem_i).wait()

        rows0 = lax.iota(jnp.int32, _LANES)
        rows1 = rows0 + _LANES
        sems = (s0, s1)

        def fire(c, g):
            vec = idx_v[pl.ds(c * 128 + (g // 2) * _LANES, _LANES)]
            half = (g % 2) * _GRP
            for j in range(_GRP):
                boff = pl.multiple_of((vec[half + j] >> 7) * 128, 128)
                pltpu.async_copy(
                    tab_hbm.at[:, pl.ds(boff, 128)],
                    blk_v.at[g % 2, j],
                    sems[g % 2],
                )

        def drain_extract(c, g):
            for _ in range(_GRP):
                pltpu.make_async_copy(
                    tab_hbm.at[:, pl.ds(0, 128)],
                    blk_v.at[0, 0],
                    sems[g % 2],
                ).wait()
            vec = idx_v[pl.ds(c * 128 + (g // 2) * _LANES, _LANES)]
            half = (g % 2) * _GRP
            lanes = jnp.bitwise_and(vec, 127)
            for j in range(_GRP):
                lane = jnp.broadcast_to(lanes[half + j], (_LANES,))
                pos = jnp.broadcast_to(g * _GRP + j, (_LANES,))
                c0 = plsc.load_gather(blk_v.at[g % 2, j], [rows0, lane])
                c1 = plsc.load_gather(blk_v.at[g % 2, j], [rows1, lane])
                plsc.store_scatter(col_v, [rows0, pos], c0)
                plsc.store_scatter(col_v, [rows1, pos], c1)

        def chunk_body(c, carry):
            fire(c, 0)
            for g in range(n_grp):
                if g + 1 < n_grp:
                    fire(c, g + 1)
                drain_extract(c, g)
            ob = pl.multiple_of(base + c * 128, 128)
            pltpu.sync_copy(col_v, out_hbm.at[:, pl.ds(ob, 128)])
            return carry

        lax.fori_loop(0, n_chunks, chunk_body, 0)

    return k


def kernel(indices, table):
    B, = indices.shape
    V, D = table.shape
    lookup = _make_lookup(V, D, B)
    out_t = lookup(table.T, indices.astype(jnp.int32))
    return out_t.T


# speedup vs baseline: 1.0045x; 1.0015x over previous
import functools

import jax
import jax.numpy as jnp
from jax import lax
from jax.experimental import pallas as pl
from jax.experimental.pallas import tpu as pltpu
from jax.experimental.pallas import tpu_sc as plsc

_LANES = 16
_GRP = 8


def _make_lookup(V, D, B):
    info = plsc.get_sparse_core_info()
    NC, NS = info.num_cores, info.num_subcores
    NW = NC * NS
    assert B % (128 * NW) == 0
    assert D == 2 * _LANES
    b_per_w = B // NW
    n_chunks = b_per_w // 128
    n_grp = 128 // _GRP

    mesh = plsc.VectorSubcoreMesh(core_axis_name="c", subcore_axis_name="s")

    @functools.partial(
        pl.kernel,
        mesh=mesh,
        out_type=jax.ShapeDtypeStruct((D, B), jnp.float32),
        scratch_types=[
            pltpu.VMEM((b_per_w,), jnp.int32),
            pltpu.VMEM((2, _GRP, D, 128), jnp.float32),
            pltpu.VMEM((D, 128), jnp.float32),
            pltpu.SemaphoreType.DMA,
            pltpu.SemaphoreType.DMA,
            pltpu.SemaphoreType.DMA,
        ],
        compiler_params=pltpu.CompilerParams(
            use_tc_tiling_on_sc=True, needs_layout_passes=False
        ),
    )
    def k(tab_hbm, idx_hbm, out_hbm, idx_v, blk_v, col_v, sem_i, s0, s1):
        wid = lax.axis_index("s") * NC + lax.axis_index("c")
        base = wid * b_per_w
        pltpu.async_copy(idx_hbm.at[pl.ds(base, b_per_w)], idx_v, sem_i).wait()

        rows0 = lax.iota(jnp.int32, _LANES)
        rows1 = rows0 + _LANES
        sems = (s0, s1)

        def fire(c, g):
            vec = idx_v[pl.ds(c * 128 + (g // 2) * _LANES, _LANES)]
            half = (g % 2) * _GRP
            for j in range(_GRP):
                boff = pl.multiple_of((vec[half + j] >> 7) * 128, 128)
                pltpu.async_copy(
                    tab_hbm.at[:, pl.ds(boff, 128)],
                    blk_v.at[g % 2, j],
                    sems[g % 2],
                )

        def drain_extract(c, g):
            for _ in range(_GRP):
                pltpu.make_async_copy(
                    tab_hbm.at[:, pl.ds(0, 128)],
                    blk_v.at[0, 0],
                    sems[g % 2],
                ).wait()
            vec = idx_v[pl.ds(c * 128 + (g // 2) * _LANES, _LANES)]
            half = (g % 2) * _GRP
            lanes = jnp.bitwise_and(vec, 127)
            for j in range(_GRP):
                lane = jnp.broadcast_to(lanes[half + j], (_LANES,))
                pos = jnp.broadcast_to(g * _GRP + j, (_LANES,))
                c0 = plsc.load_gather(blk_v.at[g % 2, j], [rows0, lane])
                c1 = plsc.load_gather(blk_v.at[g % 2, j], [rows1, lane])
                plsc.store_scatter(col_v, [rows0, pos], c0)
                plsc.store_scatter(col_v, [rows1, pos], c1)

        def chunk_body(c, carry):
            fire(c, 0)
            for g in range(n_grp):
                if g + 1 < n_grp:
                    fire(c, g + 1)
                drain_extract(c, g)
            ob = pl.multiple_of(base + c * 128, 128)
            pltpu.sync_copy(col_v, out_hbm.at[:, pl.ds(ob, 128)])
            return carry

        lax.fori_loop(0, n_chunks, chunk_body, 0)

    return k


def kernel(indices, table):
    B, = indices.shape
    V, D = table.shape
    lookup = _make_lookup(V, D, B)
    out_t = lookup(table.T, indices.astype(jnp.int32))
    return out_t.T
